# Initial kernel scaffold; baseline (speedup 1.0000x reference)
import jax
import jax.numpy as jnp
from jax.experimental import pallas as pl


def kernel(features, same_index, diff_index, params):
    raise NotImplementedError("write your pallas kernel here")


# speedup vs baseline: 16.1830x; 16.1830x over previous
import functools

import jax
import jax.numpy as jnp
from jax import lax
from jax.experimental import pallas as pl
from jax.experimental.pallas import tpu as pltpu
from jax.experimental.pallas import tpu_sc as plsc

N = 10000
E = 320000
DIN = 128
H = 20
NEG_SLOPE = 0.01
LANES = 16
NTILES = 32
C = E // NTILES
NVEC = C // LANES
NPAD = 10240
PVEC = NPAD // LANES
INV_SQRT_H = 1.0 / (float(H) ** 0.5)


def _edge_body(qkvT_hbm, edges_hbm, out_hbm, dst_r, src_r, s_r, rowa_r, rowb_r,
               part_r):
    cid = lax.axis_index("c")
    sid = lax.axis_index("s")
    wid = sid * 2 + cid

    def zero_s(j, carry):
        s_r[pl.ds(j * LANES, LANES)] = jnp.zeros((LANES,), jnp.float32)
        return carry

    def zero_part(j, carry):
        part_r[pl.ds(j * LANES, LANES)] = jnp.zeros((LANES,), jnp.float32)
        return carry

    for g in range(2):
        pltpu.sync_copy(edges_hbm.at[g, 1, wid, 0], dst_r)
        pltpu.sync_copy(edges_hbm.at[g, 0, wid, 0], src_r)

        lax.fori_loop(0, NVEC, zero_s, 0, unroll=4)

        def score_feature(f, carry):
            pltpu.sync_copy(qkvT_hbm.at[g, 0, f, 0], rowa_r)
            pltpu.sync_copy(qkvT_hbm.at[g, 1, f, 0], rowb_r)

            def accum(j, inner):
                sl = pl.ds(j * LANES, LANES)
                qv = plsc.load_gather(rowa_r, [dst_r[sl]])
                kv = plsc.load_gather(rowb_r, [src_r[sl]])
                s_r[sl] = s_r[sl] + qv * kv
                return inner

            lax.fori_loop(0, NVEC, accum, 0, unroll=4)
            return carry

        lax.fori_loop(0, H, score_feature, 0)

        def expo(j, carry):
            sl = pl.ds(j * LANES, LANES)
            s_r[sl] = jnp.exp(s_r[sl] * INV_SQRT_H)
            return carry

        lax.fori_loop(0, NVEC, expo, 0, unroll=4)

        lax.fori_loop(0, PVEC, zero_part, 0, unroll=4)

        def denom(j, carry):
            sl = pl.ds(j * LANES, LANES)
            plsc.addupdate_scatter(part_r, [dst_r[sl]], s_r[sl])
            return carry

        lax.fori_loop(0, NVEC, denom, 0, unroll=4)
        pltpu.sync_copy(part_r, out_hbm.at[g, wid, H, 0])

        def agg_feature(f, carry):
            pltpu.sync_copy(qkvT_hbm.at[g, 2, f, 0], rowa_r)
            lax.fori_loop(0, PVEC, zero_part, 0, unroll=4)

            def accum(j, inner):
                sl = pl.ds(j * LANES, LANES)
                vv = plsc.load_gather(rowa_r, [src_r[sl]])
                plsc.addupdate_scatter(part_r, [dst_r[sl]], s_r[sl] * vv)
                return inner

            lax.fori_loop(0, NVEC, accum, 0, unroll=4)
            pltpu.sync_copy(part_r, out_hbm.at[g, wid, f, 0])
            return carry

        lax.fori_loop(0, H, agg_feature, 0)


_edge_call = pl.kernel(
    _edge_body,
    out_type=jax.ShapeDtypeStruct((2, NTILES, H + 1, 1, NPAD), jnp.float32),
    mesh=plsc.VectorSubcoreMesh(core_axis_name="c", subcore_axis_name="s",
                                num_cores=2, num_subcores=16),
    scratch_types=[
        pltpu.VMEM((C,), jnp.int32),
        pltpu.VMEM((C,), jnp.int32),
        pltpu.VMEM((C,), jnp.float32),
        pltpu.VMEM((NPAD,), jnp.float32),
        pltpu.VMEM((NPAD,), jnp.float32),
        pltpu.VMEM((NPAD,), jnp.float32),
    ],
    compiler_params=pltpu.CompilerParams(needs_layout_passes=False),
)


def _proj_body(xT_ref, w_ref, b_ref, qkvT_ref, skipT_ref):
    xT = xT_ref[...]
    for st in range(2):
        for m in range(4):
            w = w_ref[st, m]
            res = lax.dot_general(w, xT, (((0,), (0,)), ((), ())),
                                  preferred_element_type=jnp.float32)
            res = res + b_ref[st, m][:, None]
            if m < 3:
                qkvT_ref[st, m] = res
            else:
                skipT_ref[st] = res


def _project(xT, w, b):
    din = xT.shape[0]
    return pl.pallas_call(
        _proj_body,
        out_shape=(
            jax.ShapeDtypeStruct((2, 3, H, NPAD), jnp.float32),
            jax.ShapeDtypeStruct((2, H, NPAD), jnp.float32),
        ),
    )(xT, w, b)


def _reduce_body(parts_ref, out_ref):
    @pl.when(pl.program_id(0) == 0)
    def _():
        out_ref[...] = jnp.zeros_like(out_ref)

    out_ref[...] += parts_ref[:, 0]


def _reduce_partials(parts):
    return pl.pallas_call(
        _reduce_body,
        grid=(NTILES,),
        in_specs=[pl.BlockSpec((2, 1, H + 1, NPAD), lambda i: (0, i, 0, 0))],
        out_specs=pl.BlockSpec((2, H + 1, NPAD), lambda i: (0, 0, 0)),
        out_shape=jax.ShapeDtypeStruct((2, H + 1, NPAD), jnp.float32),
    )(parts)


def _combine_body(red_ref, skipT_ref, wgt_ref, gamma_ref, beta_ref, xT_ref,
                  *, layer):
    xs = []
    for st in range(2):
        u = red_ref[st, :H, :]
        d = red_ref[st, H:H + 1, :]
        xs.append(u / (d + 1e-16) + skipT_ref[st])
    wgt = wgt_ref[...]
    w1 = wgt[0, layer]
    w2 = wgt[1, layer]
    s = w1 + w2
    xpre = (w1 / s) * xs[0] + (w2 / s) * xs[1]

    node = lax.broadcasted_iota(jnp.int32, (1, NPAD), 1)
    valid = node < N
    xm = jnp.where(valid, xpre, 0.0)
    mean = jnp.sum(xm, axis=1, keepdims=True) / N
    var = jnp.sum(xm * xm, axis=1, keepdims=True) / N - mean * mean
    xn = (xpre - mean) * lax.rsqrt(var + 1e-5)
    xn = xn * gamma_ref[...] + beta_ref[...]
    xT_ref[...] = jnp.where(xn >= 0, xn, NEG_SLOPE * xn)


def _combine(red, skipT, wgt, gamma, beta, layer):
    return pl.pallas_call(
        functools.partial(_combine_body, layer=layer),
        out_shape=jax.ShapeDtypeStruct((H, NPAD), jnp.float32),
    )(red, skipT, wgt, gamma, beta)


def _head_body(x0_ref, x1_ref, x2_ref, x3_ref, wout_ref, bout_ref, out_ref):
    acc = bout_ref[...]
    for i, xr in enumerate((x0_ref, x1_ref, x2_ref, x3_ref)):
        acc = acc + lax.dot_general(xr[...], wout_ref[pl.ds(H * i, H), :],
                                    (((0,), (0,)), ((), ())),
                                    preferred_element_type=jnp.float32)
    out_ref[...] = acc


def _head(fcs, wout, bout):
    return pl.pallas_call(
        _head_body,
        out_shape=jax.ShapeDtypeStruct((NPAD, 2), jnp.float32),
    )(fcs[0], fcs[1], fcs[2], fcs[3], wout, bout)


@jax.jit
def kernel(features, same_index, diff_index, params):
    featT = jnp.zeros((DIN, NPAD), jnp.float32).at[:, :N].set(features.T)
    edges = jnp.stack([same_index, diff_index]).astype(jnp.int32).reshape(2, 2, NTILES, 1, C)

    wmats = []
    bvecs = []
    for l in range(4):
        wmats.append(jnp.stack([
            jnp.stack([params[st][l][k] for k in ('Wq', 'Wk', 'Wv', 'Wskip')])
            for st in ('c1', 'c2')]))
        bvecs.append(jnp.stack([
            jnp.stack([params[st][l][k] for k in ('bq', 'bk', 'bv', 'bskip')])
            for st in ('c1', 'c2')]))
    wgt = jnp.stack([params['w1'], params['w2']])

    xT = featT
    fcs = []
    for l in range(4):
        qkvT, skipT = _project(xT, wmats[l], bvecs[l])
        parts = _edge_call(qkvT[:, :, :, None, :], edges)
        red = _reduce_partials(parts[:, :, :, 0, :])
        xT = _combine(red, skipT, wgt,
                      params['bn'][l]['gamma'][:, None],
                      params['bn'][l]['beta'][:, None], l)
        fcs.append(xT)

    out = _head(fcs, params['Wout'], params['bout'][None, :])
    return out[:N]
